# baseline (device time: 1398476 ns/iter reference)
import jax
import jax.numpy as jnp
from jax import lax
from jax.experimental import pallas as pl
from jax.experimental.pallas import tpu as pltpu

N_DEV = 16
M_CHUNK = 256


def _sigma(r):
    r = r % N_DEV
    col = r // 4
    t = r % 4
    z = jnp.where(col % 2 == 0, t, 3 - t)
    return 4 * z + col


def _ring_pos(p):
    col = p % 4
    z = p // 4
    t = jnp.where(col % 2 == 0, z, 3 - z)
    return 4 * col + t


def kernel(x, w_mat, scale_x, scale_w):
    m, k = x.shape
    k2, n = w_mat.shape
    assert m == N_DEV * M_CHUNK

    def body(x_ref, w_ref, sx_ref, sw_ref, out_ref,
             comm_ref, send_sems, recv_sems, credit_sem):
        p = lax.axis_index("i")
        r = _ring_pos(p)
        dst = _sigma(r + 1)
        src = _sigma(r - 1)

        barrier_sem = pltpu.get_barrier_semaphore()
        for nbr in (src, dst):
            pl.semaphore_signal(
                barrier_sem, inc=1,
                device_id=(nbr,), device_id_type=pl.DeviceIdType.MESH,
            )
        pl.semaphore_wait(barrier_sem, 2)

        w_bf = w_ref[:, :].astype(jnp.bfloat16)

        def partial_for(chunk):
            rows = x_ref[pl.ds(chunk * M_CHUNK, M_CHUNK), :].astype(jnp.bfloat16)
            return jnp.dot(rows, w_bf, preferred_element_type=jnp.float32)

        comm_ref[0, :, :] = partial_for(_sigma(r - 1))

        for s in range(N_DEV - 1):
            send_slot = s % 2
            recv_slot = (s + 1) % 2
            if s >= 2:
                pl.semaphore_wait(credit_sem, 1)
            rdma = pltpu.make_async_remote_copy(
                src_ref=comm_ref.at[send_slot],
                dst_ref=comm_ref.at[recv_slot],
                send_sem=send_sems.at[send_slot],
                recv_sem=recv_sems.at[recv_slot],
                device_id=(dst,),
                device_id_type=pl.DeviceIdType.MESH,
            )
            rdma.start()
            acc = partial_for(_sigma(r - s - 2))
            rdma.wait()
            if s <= 12:
                pl.semaphore_signal(
                    credit_sem, inc=1,
                    device_id=(src,), device_id_type=pl.DeviceIdType.MESH,
                )
            if s < N_DEV - 2:
                comm_ref[recv_slot, :, :] = comm_ref[recv_slot, :, :] + acc
            else:
                scale = sx_ref[0] * sw_ref[0]
                out_ref[:, :] = (comm_ref[recv_slot, :, :] + acc) * scale

    return pl.pallas_call(
        body,
        out_shape=jax.ShapeDtypeStruct((M_CHUNK, n), jnp.float32),
        in_specs=[
            pl.BlockSpec(memory_space=pltpu.VMEM),
            pl.BlockSpec(memory_space=pltpu.VMEM),
            pl.BlockSpec(memory_space=pltpu.SMEM),
            pl.BlockSpec(memory_space=pltpu.SMEM),
        ],
        out_specs=pl.BlockSpec(memory_space=pltpu.VMEM),
        scratch_shapes=[
            pltpu.VMEM((2, M_CHUNK, n), jnp.float32),
            pltpu.SemaphoreType.DMA((2,)),
            pltpu.SemaphoreType.DMA((2,)),
            pltpu.SemaphoreType.REGULAR,
        ],
        compiler_params=pltpu.CompilerParams(collective_id=0),
    )(x, w_mat, scale_x, scale_w)


# device time: 726839 ns/iter; 1.9241x vs baseline; 1.9241x over previous
import jax
import jax.numpy as jnp
from jax import lax
from jax.experimental import pallas as pl
from jax.experimental.pallas import tpu as pltpu

N_DEV = 16
M_CHUNK = 256


def _sigma(r):
    r = r % N_DEV
    col = r // 4
    t = r % 4
    z = jnp.where(col % 2 == 0, t, 3 - t)
    return 4 * z + col


def _ring_pos(p):
    col = p % 4
    z = p // 4
    t = jnp.where(col % 2 == 0, z, 3 - z)
    return 4 * col + t


def kernel(x, w_mat, scale_x, scale_w):
    m, k = x.shape
    k2, n = w_mat.shape
    nh = n // 2
    assert m == N_DEV * M_CHUNK

    def body(x_ref, w_ref, sx_ref, sw_ref, out_ref,
             comm_f, comm_b, send_sems_f, recv_sems_f, send_sems_b,
             recv_sems_b, credit_f, credit_b):
        p = lax.axis_index("i")
        r = _ring_pos(p)
        nxt = _sigma(r + 1)
        prv = _sigma(r - 1)

        barrier_sem = pltpu.get_barrier_semaphore()
        for nbr in (prv, nxt):
            pl.semaphore_signal(
                barrier_sem, inc=1,
                device_id=(nbr,), device_id_type=pl.DeviceIdType.MESH,
            )
        pl.semaphore_wait(barrier_sem, 2)

        w_bf = w_ref[:, :].astype(jnp.bfloat16)

        def partial_for(chunk, lo, width):
            rows = x_ref[pl.ds(chunk * M_CHUNK, M_CHUNK), :].astype(jnp.bfloat16)
            return jnp.dot(rows, w_bf[:, lo:lo + width],
                           preferred_element_type=jnp.float32)

        comm_f[0, :, :] = partial_for(_sigma(r - 1), 0, nh)
        comm_b[0, :, :] = partial_for(_sigma(r + 1), nh, nh)

        for s in range(N_DEV - 1):
            send_slot = s % 2
            recv_slot = (s + 1) % 2
            if s >= 2:
                pl.semaphore_wait(credit_f, 1)
                pl.semaphore_wait(credit_b, 1)
            rdma_f = pltpu.make_async_remote_copy(
                src_ref=comm_f.at[send_slot],
                dst_ref=comm_f.at[recv_slot],
                send_sem=send_sems_f.at[send_slot],
                recv_sem=recv_sems_f.at[recv_slot],
                device_id=(nxt,),
                device_id_type=pl.DeviceIdType.MESH,
            )
            rdma_b = pltpu.make_async_remote_copy(
                src_ref=comm_b.at[send_slot],
                dst_ref=comm_b.at[recv_slot],
                send_sem=send_sems_b.at[send_slot],
                recv_sem=recv_sems_b.at[recv_slot],
                device_id=(prv,),
                device_id_type=pl.DeviceIdType.MESH,
            )
            rdma_f.start()
            rdma_b.start()
            acc_f = partial_for(_sigma(r - s - 2), 0, nh)
            acc_b = partial_for(_sigma(r + s + 2), nh, nh)
            rdma_f.wait()
            rdma_b.wait()
            if s <= 12:
                pl.semaphore_signal(
                    credit_f, inc=1,
                    device_id=(prv,), device_id_type=pl.DeviceIdType.MESH,
                )
                pl.semaphore_signal(
                    credit_b, inc=1,
                    device_id=(nxt,), device_id_type=pl.DeviceIdType.MESH,
                )
            if s < N_DEV - 2:
                comm_f[recv_slot, :, :] = comm_f[recv_slot, :, :] + acc_f
                comm_b[recv_slot, :, :] = comm_b[recv_slot, :, :] + acc_b
            else:
                scale = sx_ref[0] * sw_ref[0]
                out_ref[:, :nh] = (comm_f[recv_slot, :, :] + acc_f) * scale
                out_ref[:, nh:] = (comm_b[recv_slot, :, :] + acc_b) * scale

    return pl.pallas_call(
        body,
        out_shape=jax.ShapeDtypeStruct((M_CHUNK, n), jnp.float32),
        in_specs=[
            pl.BlockSpec(memory_space=pltpu.VMEM),
            pl.BlockSpec(memory_space=pltpu.VMEM),
            pl.BlockSpec(memory_space=pltpu.SMEM),
            pl.BlockSpec(memory_space=pltpu.SMEM),
        ],
        out_specs=pl.BlockSpec(memory_space=pltpu.VMEM),
        scratch_shapes=[
            pltpu.VMEM((2, M_CHUNK, nh), jnp.float32),
            pltpu.VMEM((2, M_CHUNK, nh), jnp.float32),
            pltpu.SemaphoreType.DMA((2,)),
            pltpu.SemaphoreType.DMA((2,)),
            pltpu.SemaphoreType.DMA((2,)),
            pltpu.SemaphoreType.DMA((2,)),
            pltpu.SemaphoreType.REGULAR,
            pltpu.SemaphoreType.REGULAR,
        ],
        compiler_params=pltpu.CompilerParams(collective_id=0),
    )(x, w_mat, scale_x, scale_w)


# device time: 388731 ns/iter; 3.5975x vs baseline; 1.8698x over previous
import jax
import jax.numpy as jnp
from jax import lax
from jax.experimental import pallas as pl
from jax.experimental.pallas import tpu as pltpu

N_DEV = 16
M_CHUNK = 256


def _sigma(r):
    r = r % N_DEV
    col = r // 4
    t = r % 4
    z = jnp.where(col % 2 == 0, t, 3 - t)
    return 4 * z + col


def _ring_pos(p):
    col = p % 4
    z = p // 4
    t = jnp.where(col % 2 == 0, z, 3 - z)
    return 4 * col + t


def kernel(x, w_mat, scale_x, scale_w):
    m, k = x.shape
    k2, n = w_mat.shape
    nh = n // 2
    assert m == N_DEV * M_CHUNK

    def body(x_ref, w_ref, sx_ref, sw_ref, out_ref,
             comm_f, comm_b, send_sems_f, recv_sems_f, send_sems_b,
             recv_sems_b, credit_f, credit_b):
        p = lax.axis_index("i")
        r = _ring_pos(p)
        nxt = _sigma(r + 1)
        prv = _sigma(r - 1)

        barrier_sem = pltpu.get_barrier_semaphore()
        for nbr in (prv, nxt):
            pl.semaphore_signal(
                barrier_sem, inc=1,
                device_id=(nbr,), device_id_type=pl.DeviceIdType.MESH,
            )
        pl.semaphore_wait(barrier_sem, 2)

        w_bf = w_ref[:, :].astype(jnp.bfloat16)

        def partial_for(chunk, lo, width):
            rows = x_ref[pl.ds(chunk * M_CHUNK, M_CHUNK), :].astype(jnp.bfloat16)
            return jnp.dot(rows, w_bf[:, lo:lo + width],
                           preferred_element_type=jnp.float32)

        comm_f[0, :, :] = partial_for(_sigma(r - 1), 0, nh).astype(jnp.bfloat16)
        comm_b[0, :, :] = partial_for(_sigma(r + 1), nh, nh).astype(jnp.bfloat16)

        for s in range(N_DEV - 1):
            send_slot = s % 2
            recv_slot = (s + 1) % 2
            if s >= 2:
                pl.semaphore_wait(credit_f, 1)
                pl.semaphore_wait(credit_b, 1)
            rdma_f = pltpu.make_async_remote_copy(
                src_ref=comm_f.at[send_slot],
                dst_ref=comm_f.at[recv_slot],
                send_sem=send_sems_f.at[send_slot],
                recv_sem=recv_sems_f.at[recv_slot],
                device_id=(nxt,),
                device_id_type=pl.DeviceIdType.MESH,
            )
            rdma_b = pltpu.make_async_remote_copy(
                src_ref=comm_b.at[send_slot],
                dst_ref=comm_b.at[recv_slot],
                send_sem=send_sems_b.at[send_slot],
                recv_sem=recv_sems_b.at[recv_slot],
                device_id=(prv,),
                device_id_type=pl.DeviceIdType.MESH,
            )
            rdma_f.start()
            rdma_b.start()
            acc_f = partial_for(_sigma(r - s - 2), 0, nh)
            acc_b = partial_for(_sigma(r + s + 2), nh, nh)
            rdma_f.wait()
            rdma_b.wait()
            if s <= 12:
                pl.semaphore_signal(
                    credit_f, inc=1,
                    device_id=(prv,), device_id_type=pl.DeviceIdType.MESH,
                )
                pl.semaphore_signal(
                    credit_b, inc=1,
                    device_id=(nxt,), device_id_type=pl.DeviceIdType.MESH,
                )
            if s < N_DEV - 2:
                comm_f[recv_slot, :, :] = (
                    comm_f[recv_slot, :, :].astype(jnp.float32) + acc_f
                ).astype(jnp.bfloat16)
                comm_b[recv_slot, :, :] = (
                    comm_b[recv_slot, :, :].astype(jnp.float32) + acc_b
                ).astype(jnp.bfloat16)
            else:
                scale = sx_ref[0] * sw_ref[0]
                out_ref[:, :nh] = (
                    comm_f[recv_slot, :, :].astype(jnp.float32) + acc_f
                ) * scale
                out_ref[:, nh:] = (
                    comm_b[recv_slot, :, :].astype(jnp.float32) + acc_b
                ) * scale

    return pl.pallas_call(
        body,
        out_shape=jax.ShapeDtypeStruct((M_CHUNK, n), jnp.float32),
        in_specs=[
            pl.BlockSpec(memory_space=pltpu.VMEM),
            pl.BlockSpec(memory_space=pltpu.VMEM),
            pl.BlockSpec(memory_space=pltpu.SMEM),
            pl.BlockSpec(memory_space=pltpu.SMEM),
        ],
        out_specs=pl.BlockSpec(memory_space=pltpu.VMEM),
        scratch_shapes=[
            pltpu.VMEM((2, M_CHUNK, nh), jnp.bfloat16),
            pltpu.VMEM((2, M_CHUNK, nh), jnp.bfloat16),
            pltpu.SemaphoreType.DMA((2,)),
            pltpu.SemaphoreType.DMA((2,)),
            pltpu.SemaphoreType.DMA((2,)),
            pltpu.SemaphoreType.DMA((2,)),
            pltpu.SemaphoreType.REGULAR,
            pltpu.SemaphoreType.REGULAR,
        ],
        compiler_params=pltpu.CompilerParams(collective_id=0),
    )(x, w_mat, scale_x, scale_w)


# device time: 352822 ns/iter; 3.9637x vs baseline; 1.1018x over previous
import jax
import jax.numpy as jnp
from jax import lax
from jax.experimental import pallas as pl
from jax.experimental.pallas import tpu as pltpu

N_DEV = 16
M_CHUNK = 256


def _sigma(r):
    r = r % N_DEV
    col = r // 4
    t = r % 4
    z = jnp.where(col % 2 == 0, t, 3 - t)
    return 4 * z + col


def _ring_pos(p):
    col = p % 4
    z = p // 4
    t = jnp.where(col % 2 == 0, z, 3 - z)
    return 4 * col + t


def kernel(x, w_mat, scale_x, scale_w):
    m, k = x.shape
    k2, n = w_mat.shape
    nq = n // 4
    assert m == N_DEV * M_CHUNK

    def body(x_ref, w_ref, sx_ref, sw_ref, out_ref, *scratch):
        comms = scratch[0:4]
        send_sems = scratch[4:8]
        recv_sems = scratch[8:12]
        credits = scratch[12:16]

        p = lax.axis_index("i")
        r = _ring_pos(p)
        nxt = _sigma(r + 1)
        prv = _sigma(r - 1)
        dsts = (nxt, nxt, prv, prv)
        srcs = (prv, prv, nxt, nxt)
        col_lo = (0, nq, 2 * nq, 3 * nq)

        barrier_sem = pltpu.get_barrier_semaphore()
        for nbr in (prv, nxt):
            pl.semaphore_signal(
                barrier_sem, inc=1,
                device_id=(nbr,), device_id_type=pl.DeviceIdType.MESH,
            )
        pl.semaphore_wait(barrier_sem, 2)

        w_bf = w_ref[:, :].astype(jnp.bfloat16)

        def partial_for(chunk, ring):
            rows = x_ref[pl.ds(chunk * M_CHUNK, M_CHUNK), :].astype(jnp.bfloat16)
            return jnp.dot(rows, w_bf[:, col_lo[ring]:col_lo[ring] + nq],
                           preferred_element_type=jnp.float32)

        def chunk_at(ring, s):
            return _sigma(r - s - 2) if ring < 2 else _sigma(r + s + 2)

        def make_rdma(ring, s):
            send_slot = s % 2
            recv_slot = (s + 1) % 2
            return pltpu.make_async_remote_copy(
                src_ref=comms[ring].at[send_slot],
                dst_ref=comms[ring].at[recv_slot],
                send_sem=send_sems[ring].at[send_slot],
                recv_sem=recv_sems[ring].at[recv_slot],
                device_id=(dsts[ring],),
                device_id_type=pl.DeviceIdType.MESH,
            )

        def start(ring, s):
            if s >= 2:
                pl.semaphore_wait(credits[ring], 1)
            make_rdma(ring, s).start()

        def finish(ring, s, acc):
            recv_slot = (s + 1) % 2
            make_rdma(ring, s).wait()
            if s <= 12:
                pl.semaphore_signal(
                    credits[ring], inc=1,
                    device_id=(srcs[ring],),
                    device_id_type=pl.DeviceIdType.MESH,
                )
            total = comms[ring][recv_slot, :, :].astype(jnp.float32) + acc
            if s < N_DEV - 2:
                comms[ring][recv_slot, :, :] = total.astype(jnp.bfloat16)
            else:
                scale = sx_ref[0] * sw_ref[0]
                out_ref[:, col_lo[ring]:col_lo[ring] + nq] = total * scale

        for ring in range(4):
            first = _sigma(r - 1) if ring < 2 else _sigma(r + 1)
            comms[ring][0, :, :] = partial_for(first, ring).astype(jnp.bfloat16)
        start(0, 0)
        start(2, 0)

        for s in range(N_DEV - 1):
            start(1, s)
            start(3, s)
            acc = [partial_for(chunk_at(ring, s), ring) for ring in range(4)]
            finish(0, s, acc[0])
            finish(2, s, acc[2])
            if s < N_DEV - 2:
                start(0, s + 1)
                start(2, s + 1)
            finish(1, s, acc[1])
            finish(3, s, acc[3])

    return pl.pallas_call(
        body,
        out_shape=jax.ShapeDtypeStruct((M_CHUNK, n), jnp.float32),
        in_specs=[
            pl.BlockSpec(memory_space=pltpu.VMEM),
            pl.BlockSpec(memory_space=pltpu.VMEM),
            pl.BlockSpec(memory_space=pltpu.SMEM),
            pl.BlockSpec(memory_space=pltpu.SMEM),
        ],
        out_specs=pl.BlockSpec(memory_space=pltpu.VMEM),
        scratch_shapes=(
            [pltpu.VMEM((2, M_CHUNK, nq), jnp.bfloat16) for _ in range(4)]
            + [pltpu.SemaphoreType.DMA((2,)) for _ in range(4)]
            + [pltpu.SemaphoreType.DMA((2,)) for _ in range(4)]
            + [pltpu.SemaphoreType.REGULAR for _ in range(4)]
        ),
        compiler_params=pltpu.CompilerParams(collective_id=0),
    )(x, w_mat, scale_x, scale_w)
